# initial kernel scaffold (unmeasured)
import jax
import jax.numpy as jnp
from jax import lax
from jax.experimental import pallas as pl
from jax.experimental.pallas import tpu as pltpu


def kernel(
    x,
):
    def body(*refs):
        pass

    out_shape = jax.ShapeDtypeStruct(..., jnp.float32)
    return pl.pallas_call(body, out_shape=out_shape)(...)


# baseline (device time: 73249 ns/iter reference)
import jax
import jax.numpy as jnp
from jax import lax
from jax.experimental import pallas as pl
from jax.experimental.pallas import tpu as pltpu

N_DEV = 16


def _cycle_pos(i):
    z = i // 4
    r = i % 4
    return 4 * r + jnp.where(r % 2 == 0, z, 3 - z)


def _logical(c):
    c = c % N_DEV
    r = c // 4
    t = c % 4
    z = jnp.where(r % 2 == 0, t, 3 - t)
    return 4 * z + r


def kernel(x):
    m, n = x.shape

    def body(x_ref, out_ref, send_sems, recv_sems):
        i = lax.axis_index("i")
        cpos = _cycle_pos(i)
        right = _logical(cpos + 1)
        left = _logical(cpos - 1)

        barrier = pltpu.get_barrier_semaphore()
        for nbr in (left, right):
            pl.semaphore_signal(
                barrier, inc=1,
                device_id=(nbr,), device_id_type=pl.DeviceIdType.MESH,
            )
        pl.semaphore_wait(barrier, 2)

        out_ref[pl.ds(i * m, m), :] = x_ref[...]

        for h in range(N_DEV - 1):
            org = _logical(cpos - h)
            rdma = pltpu.make_async_remote_copy(
                src_ref=out_ref.at[pl.ds(org * m, m), :],
                dst_ref=out_ref.at[pl.ds(org * m, m), :],
                send_sem=send_sems.at[h],
                recv_sem=recv_sems.at[h],
                device_id=(right,),
                device_id_type=pl.DeviceIdType.MESH,
            )
            rdma.start()
            rdma.wait()

    return pl.pallas_call(
        body,
        out_shape=jax.ShapeDtypeStruct((N_DEV * m, n), x.dtype),
        in_specs=[pl.BlockSpec(memory_space=pltpu.VMEM)],
        out_specs=pl.BlockSpec(memory_space=pltpu.VMEM),
        scratch_shapes=[
            pltpu.SemaphoreType.DMA((N_DEV - 1,)),
            pltpu.SemaphoreType.DMA((N_DEV - 1,)),
        ],
        compiler_params=pltpu.CompilerParams(collective_id=0),
    )(x)


# device time: 32387 ns/iter; 2.2617x vs baseline; 2.2617x over previous
import jax
import jax.numpy as jnp
from jax import lax
from jax.experimental import pallas as pl
from jax.experimental.pallas import tpu as pltpu

N_DEV = 16
R_HOPS = 8
L_HOPS = 7
SUB = 4


def _cycle_pos(i):
    z = i // 4
    r = i % 4
    return 4 * r + jnp.where(r % 2 == 0, z, 3 - z)


def _logical(c):
    c = c % N_DEV
    r = c // 4
    t = c % 4
    z = jnp.where(r % 2 == 0, t, 3 - t)
    return 4 * z + r


def kernel(x):
    m, n = x.shape
    sub_m = m // SUB

    def body(x_ref, out_ref, rs_sems, rr_sems, ls_sems, lr_sems):
        i = lax.axis_index("i")
        cpos = _cycle_pos(i)
        right = _logical(cpos + 1)
        left = _logical(cpos - 1)

        barrier = pltpu.get_barrier_semaphore()
        for nbr in (left, right):
            pl.semaphore_signal(
                barrier, inc=1,
                device_id=(nbr,), device_id_type=pl.DeviceIdType.MESH,
            )
        pl.semaphore_wait(barrier, 2)

        out_ref[pl.ds(i * m, m), :] = x_ref[...]

        def copy(org, s, dev, send_sem, recv_sem, from_x=False):
            row = org * m + s * sub_m
            src = x_ref.at[pl.ds(s * sub_m, sub_m), :] if from_x else \
                out_ref.at[pl.ds(row, sub_m), :]
            return pltpu.make_async_remote_copy(
                src_ref=src,
                dst_ref=out_ref.at[pl.ds(row, sub_m), :],
                send_sem=send_sem,
                recv_sem=recv_sem,
                device_id=(dev,),
                device_id_type=pl.DeviceIdType.MESH,
            )

        def r_send(h, s):
            return copy(_logical(cpos - h), s, right,
                        rs_sems.at[h * SUB + s], rr_sems.at[h * SUB + s],
                        from_x=(h == 0))

        def l_send(h, s):
            return copy(_logical(cpos + h), s, left,
                        ls_sems.at[h * SUB + s], lr_sems.at[h * SUB + s],
                        from_x=(h == 0))

        def r_recv(h, s):
            return copy(_logical(cpos - h - 1), s, right,
                        rs_sems.at[h * SUB + s], rr_sems.at[h * SUB + s])

        def l_recv(h, s):
            return copy(_logical(cpos + h + 1), s, left,
                        ls_sems.at[h * SUB + s], lr_sems.at[h * SUB + s])

        sends = []

        for s in range(SUB):
            d = r_send(0, s); d.start(); sends.append(d)
        for s in range(SUB):
            d = l_send(0, s); d.start(); sends.append(d)

        for h in range(1, R_HOPS):
            for s in range(SUB):
                r_recv(h - 1, s).wait_recv()
                d = r_send(h, s); d.start(); sends.append(d)
            if h < L_HOPS:
                for s in range(SUB):
                    l_recv(h - 1, s).wait_recv()
                    d = l_send(h, s); d.start(); sends.append(d)

        for s in range(SUB):
            r_recv(R_HOPS - 1, s).wait_recv()
        for s in range(SUB):
            l_recv(L_HOPS - 1, s).wait_recv()

        for d in sends:
            d.wait_send()

    return pl.pallas_call(
        body,
        out_shape=jax.ShapeDtypeStruct((N_DEV * m, n), x.dtype),
        in_specs=[pl.BlockSpec(memory_space=pltpu.VMEM)],
        out_specs=pl.BlockSpec(memory_space=pltpu.VMEM),
        scratch_shapes=[
            pltpu.SemaphoreType.DMA((R_HOPS * SUB,)),
            pltpu.SemaphoreType.DMA((R_HOPS * SUB,)),
            pltpu.SemaphoreType.DMA((L_HOPS * SUB,)),
            pltpu.SemaphoreType.DMA((L_HOPS * SUB,)),
        ],
        compiler_params=pltpu.CompilerParams(collective_id=0),
    )(x)


# device time: 32350 ns/iter; 2.2643x vs baseline; 1.0011x over previous
import jax
import jax.numpy as jnp
from jax import lax
from jax.experimental import pallas as pl
from jax.experimental.pallas import tpu as pltpu

N_DEV = 16
R_HOPS = 8
L_HOPS = 7
SUB = 8


def _cycle_pos(i):
    z = i // 4
    r = i % 4
    return 4 * r + jnp.where(r % 2 == 0, z, 3 - z)


def _logical(c):
    c = c % N_DEV
    r = c // 4
    t = c % 4
    z = jnp.where(r % 2 == 0, t, 3 - t)
    return 4 * z + r


def kernel(x):
    m, n = x.shape
    sub_m = m // SUB

    def body(x_ref, out_ref, rs_sems, rr_sems, ls_sems, lr_sems):
        i = lax.axis_index("i")
        cpos = _cycle_pos(i)
        right = _logical(cpos + 1)
        left = _logical(cpos - 1)

        barrier = pltpu.get_barrier_semaphore()
        for nbr in (left, right):
            pl.semaphore_signal(
                barrier, inc=1,
                device_id=(nbr,), device_id_type=pl.DeviceIdType.MESH,
            )
        pl.semaphore_wait(barrier, 2)

        out_ref[pl.ds(i * m, m), :] = x_ref[...]

        def copy(org, s, dev, send_sem, recv_sem, from_x=False):
            row = org * m + s * sub_m
            src = x_ref.at[pl.ds(s * sub_m, sub_m), :] if from_x else \
                out_ref.at[pl.ds(row, sub_m), :]
            return pltpu.make_async_remote_copy(
                src_ref=src,
                dst_ref=out_ref.at[pl.ds(row, sub_m), :],
                send_sem=send_sem,
                recv_sem=recv_sem,
                device_id=(dev,),
                device_id_type=pl.DeviceIdType.MESH,
            )

        def r_send(h, s):
            return copy(_logical(cpos - h), s, right,
                        rs_sems.at[h * SUB + s], rr_sems.at[h * SUB + s],
                        from_x=(h == 0))

        def l_send(h, s):
            return copy(_logical(cpos + h), s, left,
                        ls_sems.at[h * SUB + s], lr_sems.at[h * SUB + s],
                        from_x=(h == 0))

        def r_recv(h, s):
            return copy(_logical(cpos - h - 1), s, right,
                        rs_sems.at[h * SUB + s], rr_sems.at[h * SUB + s])

        def l_recv(h, s):
            return copy(_logical(cpos + h + 1), s, left,
                        ls_sems.at[h * SUB + s], lr_sems.at[h * SUB + s])

        sends = []

        for s in range(SUB):
            d = r_send(0, s); d.start(); sends.append(d)
        for s in range(SUB):
            d = l_send(0, s); d.start(); sends.append(d)

        for h in range(1, R_HOPS):
            for s in range(SUB):
                r_recv(h - 1, s).wait_recv()
                d = r_send(h, s); d.start(); sends.append(d)
                if h < L_HOPS:
                    l_recv(h - 1, s).wait_recv()
                    d = l_send(h, s); d.start(); sends.append(d)

        for s in range(SUB):
            r_recv(R_HOPS - 1, s).wait_recv()
        for s in range(SUB):
            l_recv(L_HOPS - 1, s).wait_recv()

        for d in sends:
            d.wait_send()

    return pl.pallas_call(
        body,
        out_shape=jax.ShapeDtypeStruct((N_DEV * m, n), x.dtype),
        in_specs=[pl.BlockSpec(memory_space=pltpu.VMEM)],
        out_specs=pl.BlockSpec(memory_space=pltpu.VMEM),
        scratch_shapes=[
            pltpu.SemaphoreType.DMA((R_HOPS * SUB,)),
            pltpu.SemaphoreType.DMA((R_HOPS * SUB,)),
            pltpu.SemaphoreType.DMA((L_HOPS * SUB,)),
            pltpu.SemaphoreType.DMA((L_HOPS * SUB,)),
        ],
        compiler_params=pltpu.CompilerParams(collective_id=0),
    )(x)
